# Initial kernel scaffold; baseline (speedup 1.0000x reference)
import jax
import jax.numpy as jnp
from jax.experimental import pallas as pl


def kernel(x, table):
    raise NotImplementedError("write your pallas kernel here")


# speedup vs baseline: 1.4606x; 1.4606x over previous
import functools
from math import sqrt

import jax
import jax.numpy as jnp
from jax import lax
from jax.experimental import pallas as pl
from jax.experimental.pallas import tpu as pltpu
from jax.experimental.pallas import tpu_sc as plsc

D_MODEL = 512
SCALE = sqrt(512.0)
LANES = 16

NC = 2
NS = 16
NW = NC * NS

B = 4096 * 200
BPW = B // NW
CHUNK = 80
NCHUNK = BPW // CHUNK

_MESH = plsc.VectorSubcoreMesh(core_axis_name="c", subcore_axis_name="s")


@functools.partial(
    pl.kernel,
    mesh=_MESH,
    out_type=jax.ShapeDtypeStruct((B, D_MODEL), jnp.float32),
    scratch_types=[
        pltpu.VMEM((CHUNK,), jnp.int32),
        pltpu.VMEM((CHUNK, D_MODEL), jnp.float32),
        pltpu.SemaphoreType.DMA,
    ],
)
def _emb_lookup(table_hbm, idx_hbm, out_hbm, idx_v, rows_v, sem):
    wid = lax.axis_index("s") * NC + lax.axis_index("c")
    base = wid * BPW

    def chunk_body(g, carry):
        off = base + g * CHUNK
        pltpu.sync_copy(idx_hbm.at[pl.ds(off, CHUNK)], idx_v)
        pltpu.async_copy(table_hbm.at[idx_v], rows_v, sem).wait()

        def row_body(i, c):
            for j in range(D_MODEL // LANES):
                sl = pl.ds(j * LANES, LANES)
                rows_v[i, sl] = rows_v[i, sl] * SCALE
            return c

        lax.fori_loop(0, CHUNK, row_body, 0, unroll=False)
        pltpu.sync_copy(rows_v, out_hbm.at[pl.ds(off, CHUNK)])
        return carry

    lax.fori_loop(0, NCHUNK, chunk_body, 0, unroll=False)


def kernel(x, table):
    assert x.size == B and table.shape == (100000, D_MODEL)
    idx = x.reshape(-1).astype(jnp.int32)
    out = _emb_lookup(table, idx)
    return out.reshape(x.shape + (D_MODEL,))


# speedup vs baseline: 2.3211x; 1.5891x over previous
import functools
from math import sqrt

import jax
import jax.numpy as jnp
from jax import lax
from jax.experimental import pallas as pl
from jax.experimental.pallas import tpu as pltpu
from jax.experimental.pallas import tpu_sc as plsc

D_MODEL = 512
SCALE = sqrt(512.0)
LANES = 16

NC = 2
NS = 16
NW = NC * NS

B = 4096 * 200
BPW = B // NW
CHUNK = 80
NCHUNK = BPW // CHUNK
NPAIR = NCHUNK // 2

_MESH = plsc.VectorSubcoreMesh(core_axis_name="c", subcore_axis_name="s")


def _scale_rows(rows_v):
    def row_body(i, c):
        for j in range(D_MODEL // LANES):
            sl = pl.ds(j * LANES, LANES)
            rows_v[i, sl] = rows_v[i, sl] * SCALE
        return c

    lax.fori_loop(0, CHUNK, row_body, 0, unroll=False)


@functools.partial(
    pl.kernel,
    mesh=_MESH,
    out_type=jax.ShapeDtypeStruct((B, D_MODEL), jnp.float32),
    scratch_types=[
        pltpu.VMEM((NCHUNK, CHUNK), jnp.int32),
        pltpu.VMEM((CHUNK, D_MODEL), jnp.float32),
        pltpu.VMEM((CHUNK, D_MODEL), jnp.float32),
        pltpu.SemaphoreType.DMA,
        pltpu.SemaphoreType.DMA,
        pltpu.SemaphoreType.DMA,
        pltpu.SemaphoreType.DMA,
    ],
)
def _emb_lookup(table_hbm, idx_hbm, out_hbm, idx_v, rows0, rows1,
                gsem0, gsem1, osem0, osem1):
    wid = lax.axis_index("s") * NC + lax.axis_index("c")
    base = wid * BPW

    pltpu.sync_copy(idx_hbm.at[pl.ds(wid * NCHUNK, NCHUNK)], idx_v)

    pltpu.async_copy(table_hbm.at[idx_v.at[0]], rows0, gsem0)

    def pair_body(t, carry):
        g0 = 2 * t
        @pl.when(t > 0)
        def _():
            pltpu.make_async_copy(rows1, out_hbm.at[pl.ds(0, CHUNK)], osem1).wait()

        pltpu.async_copy(table_hbm.at[idx_v.at[g0 + 1]], rows1, gsem1)

        pltpu.make_async_copy(table_hbm.at[idx_v.at[g0]], rows0, gsem0).wait()
        _scale_rows(rows0)
        pltpu.async_copy(rows0, out_hbm.at[pl.ds(base + g0 * CHUNK, CHUNK)], osem0)

        @pl.when(t < NPAIR - 1)
        def _():
            pltpu.make_async_copy(rows0, out_hbm.at[pl.ds(0, CHUNK)], osem0).wait()
            pltpu.async_copy(table_hbm.at[idx_v.at[g0 + 2]], rows0, gsem0)

        pltpu.make_async_copy(table_hbm.at[idx_v.at[g0 + 1]], rows1, gsem1).wait()
        _scale_rows(rows1)
        pltpu.async_copy(rows1, out_hbm.at[pl.ds(base + (g0 + 1) * CHUNK, CHUNK)], osem1)
        return carry

    lax.fori_loop(0, NPAIR, pair_body, 0, unroll=False)

    pltpu.make_async_copy(rows0, out_hbm.at[pl.ds(0, CHUNK)], osem0).wait()
    pltpu.make_async_copy(rows1, out_hbm.at[pl.ds(0, CHUNK)], osem1).wait()


def kernel(x, table):
    assert x.size == B and table.shape == (100000, D_MODEL)
    idx = x.reshape(B // CHUNK, CHUNK).astype(jnp.int32)
    out = _emb_lookup(table, idx)
    return out.reshape(x.shape + (D_MODEL,))


# speedup vs baseline: 2.3241x; 1.0013x over previous
import functools
from math import sqrt

import jax
import jax.numpy as jnp
from jax import lax
from jax.experimental import pallas as pl
from jax.experimental.pallas import tpu as pltpu
from jax.experimental.pallas import tpu_sc as plsc

D_MODEL = 512
SCALE = sqrt(512.0)
LANES = 16

NC = 2
NS = 16
NW = NC * NS

B = 4096 * 200
BPW = B // NW
CHUNK = 64
NCHUNK = BPW // CHUNK
NBUF = 3

_MESH = plsc.VectorSubcoreMesh(core_axis_name="c", subcore_axis_name="s")


def _scale_rows(rows_v):
    def row_body(i, c):
        for j in range(D_MODEL // LANES):
            sl = pl.ds(j * LANES, LANES)
            rows_v[i, sl] = rows_v[i, sl] * SCALE
        return c

    lax.fori_loop(0, CHUNK, row_body, 0, unroll=False)


@functools.partial(
    pl.kernel,
    mesh=_MESH,
    out_type=jax.ShapeDtypeStruct((B, D_MODEL), jnp.float32),
    scratch_types=[
        pltpu.VMEM((NCHUNK // 2, 2 * CHUNK), jnp.int32),
        pltpu.VMEM((CHUNK, D_MODEL), jnp.float32),
        pltpu.VMEM((CHUNK, D_MODEL), jnp.float32),
        pltpu.VMEM((CHUNK, D_MODEL), jnp.float32),
        pltpu.SemaphoreType.DMA,
        pltpu.SemaphoreType.DMA,
        pltpu.SemaphoreType.DMA,
        pltpu.SemaphoreType.DMA,
        pltpu.SemaphoreType.DMA,
        pltpu.SemaphoreType.DMA,
    ],
)
def _emb_lookup(table_hbm, idx_hbm, out_hbm, idx_v, rows0, rows1, rows2,
                gsem0, gsem1, gsem2, osem0, osem1, osem2):
    rows = (rows0, rows1, rows2)
    gsems = (gsem0, gsem1, gsem2)
    osems = (osem0, osem1, osem2)

    wid = lax.axis_index("s") * NC + lax.axis_index("c")
    base = wid * BPW

    pltpu.sync_copy(idx_hbm.at[pl.ds(wid * (NCHUNK // 2), NCHUNK // 2)], idx_v)

    def idx_chunk(g):
        return idx_v.at[g // 2, pl.ds(lax.rem(g, 2) * CHUNK, CHUNK)]

    def step(g, carry):
        @pl.when(g < NCHUNK)
        def _():
            gb = lax.rem(g, NBUF)
            for b in range(NBUF):
                @pl.when(gb == b)
                def _():
                    @pl.when(g >= NBUF)
                    def _():
                        pltpu.make_async_copy(
                            rows[b], out_hbm.at[pl.ds(0, CHUNK)], osems[b]).wait()
                    pltpu.async_copy(table_hbm.at[idx_chunk(g)], rows[b], gsems[b])

        @pl.when(g >= 2)
        def _():
            p = g - 2
            pb = lax.rem(p, NBUF)
            for b in range(NBUF):
                @pl.when(pb == b)
                def _():
                    pltpu.make_async_copy(
                        table_hbm.at[idx_chunk(p)], rows[b], gsems[b]).wait()
                    _scale_rows(rows[b])
                    pltpu.async_copy(
                        rows[b], out_hbm.at[pl.ds(base + p * CHUNK, CHUNK)], osems[b])

        return carry

    lax.fori_loop(0, NCHUNK + 2, step, 0, unroll=False)

    for b in range(NBUF):
        pltpu.make_async_copy(rows[b], out_hbm.at[pl.ds(0, CHUNK)], osems[b]).wait()


def kernel(x, table):
    assert x.size == B and table.shape == (100000, D_MODEL)
    idx = x.reshape(B // (2 * CHUNK), 2 * CHUNK).astype(jnp.int32)
    out = _emb_lookup(table, idx)
    return out.reshape(x.shape + (D_MODEL,))
